# baseline (speedup 1.0000x reference)
import functools

import jax
import jax.numpy as jnp
from jax.experimental import pallas as pl
from jax.experimental.pallas import tpu as pltpu

B, C, H_p, W_p = 2, 192, 24, 24
H_img, W_img = 384, 384
NUM_SEG = 128
N_PATCH = H_p * W_p
N_PIX = H_img * W_img
PIX_BLK = 24576
ROWS_BLK = PIX_BLK // W_img
N_BLK = N_PIX // PIX_BLK


def _means_body(feat_ref, seg_ref, out_ref):
    seg = jnp.clip(seg_ref[0], 0, NUM_SEG - 1)
    iota = jax.lax.broadcasted_iota(jnp.int32, (NUM_SEG, N_PATCH), 0)
    onehot = (iota == seg).astype(jnp.float32)
    sums_t = jax.lax.dot_general(
        feat_ref[0], onehot,
        dimension_numbers=(((1,), (1,)), ((), ())),
        preferred_element_type=jnp.float32,
        precision=jax.lax.Precision.HIGHEST)
    counts = jnp.sum(onehot, axis=1)
    means = sums_t / jnp.maximum(counts, 1.0)[None, :]
    out_ref[0] = means.astype(jnp.bfloat16)


def _paint_body(seg_ref, means_ref, out_ref):
    seg = seg_ref[0]
    iota = jax.lax.broadcasted_iota(jnp.int16, (NUM_SEG, PIX_BLK), 0)
    onehot = (iota == seg).astype(jnp.bfloat16)
    out = jax.lax.dot_general(
        means_ref[0], onehot,
        dimension_numbers=(((1,), (0,)), ((), ())),
        preferred_element_type=jnp.float32)
    out_ref[0] = out.reshape(C, ROWS_BLK, W_img)


@jax.jit
def kernel(F_semantic_patches, segmentation_mask):
    feat = F_semantic_patches.reshape(B, C, N_PATCH)
    seg_small = segmentation_mask[:, ::16, ::16].reshape(B, 1, N_PATCH)

    means_t = pl.pallas_call(
        _means_body,
        grid=(B,),
        in_specs=[
            pl.BlockSpec((1, C, N_PATCH), lambda b: (b, 0, 0)),
            pl.BlockSpec((1, 1, N_PATCH), lambda b: (b, 0, 0)),
        ],
        out_specs=pl.BlockSpec((1, C, NUM_SEG), lambda b: (b, 0, 0)),
        out_shape=jax.ShapeDtypeStruct((B, C, NUM_SEG), jnp.bfloat16),
    )(feat, seg_small)

    seg_i16 = jnp.clip(segmentation_mask, 0, NUM_SEG - 1).astype(jnp.int16)
    seg_blk = seg_i16.reshape(B * N_BLK, 1, PIX_BLK)
    painted = pl.pallas_call(
        _paint_body,
        grid=(B * N_BLK,),
        in_specs=[
            pl.BlockSpec((1, 1, PIX_BLK), lambda i: (i, 0, 0)),
            pl.BlockSpec((1, C, NUM_SEG), lambda i: (i // N_BLK, 0, 0)),
        ],
        out_specs=pl.BlockSpec(
            (1, C, ROWS_BLK, W_img), lambda i: (i // N_BLK, 0, i % N_BLK, 0)),
        out_shape=jax.ShapeDtypeStruct((B, C, H_img, W_img), jnp.float32),
        compiler_params=pltpu.CompilerParams(vmem_limit_bytes=110 * 1024 * 1024),
    )(seg_blk, means_t)

    return painted


# speedup vs baseline: 1.0192x; 1.0192x over previous
import functools

import jax
import jax.numpy as jnp
from jax.experimental import pallas as pl

B, C, H_p, W_p = 2, 192, 24, 24
H_img, W_img = 384, 384
NUM_SEG = 128
N_PATCH = H_p * W_p
N_PIX = H_img * W_img
PIX_BLK = 18432
ROWS_BLK = PIX_BLK // W_img
N_BLK = N_PIX // PIX_BLK


def _means_body(feat_ref, seg_ref, out_ref):
    seg = jnp.clip(seg_ref[0], 0, NUM_SEG - 1)
    iota = jax.lax.broadcasted_iota(jnp.int32, (NUM_SEG, N_PATCH), 0)
    onehot = (iota == seg).astype(jnp.float32)
    sums_t = jax.lax.dot_general(
        feat_ref[0], onehot,
        dimension_numbers=(((1,), (1,)), ((), ())),
        preferred_element_type=jnp.float32,
        precision=jax.lax.Precision.HIGHEST)
    counts = jnp.sum(onehot, axis=1)
    means = sums_t / jnp.maximum(counts, 1.0)[None, :]
    out_ref[0] = means.astype(jnp.bfloat16)


def _paint_body(seg_ref, means_ref, out_ref):
    seg = seg_ref[0]
    iota = jax.lax.broadcasted_iota(jnp.int16, (NUM_SEG, PIX_BLK), 0)
    onehot = (iota == seg).astype(jnp.bfloat16)
    out = jax.lax.dot_general(
        means_ref[0], onehot,
        dimension_numbers=(((1,), (0,)), ((), ())),
        preferred_element_type=jnp.float32)
    out_ref[0] = out.reshape(C, ROWS_BLK, W_img)


@jax.jit
def kernel(F_semantic_patches, segmentation_mask):
    feat = F_semantic_patches.reshape(B, C, N_PATCH)
    seg_small = segmentation_mask[:, ::16, ::16].reshape(B, 1, N_PATCH)

    means_t = pl.pallas_call(
        _means_body,
        grid=(B,),
        in_specs=[
            pl.BlockSpec((1, C, N_PATCH), lambda b: (b, 0, 0)),
            pl.BlockSpec((1, 1, N_PATCH), lambda b: (b, 0, 0)),
        ],
        out_specs=pl.BlockSpec((1, C, NUM_SEG), lambda b: (b, 0, 0)),
        out_shape=jax.ShapeDtypeStruct((B, C, NUM_SEG), jnp.bfloat16),
    )(feat, seg_small)

    seg_i16 = jnp.clip(segmentation_mask, 0, NUM_SEG - 1).astype(jnp.int16)
    seg_blk = seg_i16.reshape(B * N_BLK, 1, PIX_BLK)
    painted = pl.pallas_call(
        _paint_body,
        grid=(B * N_BLK,),
        in_specs=[
            pl.BlockSpec((1, 1, PIX_BLK), lambda i: (i, 0, 0)),
            pl.BlockSpec((1, C, NUM_SEG), lambda i: (i // N_BLK, 0, 0)),
        ],
        out_specs=pl.BlockSpec(
            (1, C, ROWS_BLK, W_img), lambda i: (i // N_BLK, 0, i % N_BLK, 0)),
        out_shape=jax.ShapeDtypeStruct((B, C, H_img, W_img), jnp.float32),
    )(seg_blk, means_t)

    return painted
